# baseline (speedup 1.0000x reference)
import functools

import jax
import jax.numpy as jnp
from jax import lax
from jax.experimental import pallas as pl
from jax.experimental.pallas import tpu as pltpu
from jax.experimental.pallas import tpu_sc as plsc

B = 16384
F = 32
Q = F // 2
L = 16
NC = 2
NS = 16
NW = NC * NS
B_PER_W = B // NW
C = 128
NROWS = 1000000
RB = 65536
NBLK = 16
TOT = 4 * NBLK * 4 * RB

_mesh = plsc.VectorSubcoreMesh(core_axis_name="c", subcore_axis_name="s")


def _linearize(table_t):
    def _rne_bf16(u):
        return (u + 0x7FFF + ((u >> 16) & 1)) >> 16

    def body(x_ref, o_ref):
        for qq in range(4):
            lo = lax.bitcast_convert_type(x_ref[2 * qq, :], jnp.uint32)
            hi = lax.bitcast_convert_type(x_ref[2 * qq + 1, :], jnp.uint32)
            word = _rne_bf16(lo) | (_rne_bf16(hi) << 16)
            o_ref[pl.ds(qq * RB, RB)] = lax.bitcast_convert_type(
                word, jnp.int32)

    return pl.pallas_call(
        body,
        grid=(4, NBLK),
        in_specs=[pl.BlockSpec((8, RB), lambda a, b: (a, b))],
        out_specs=pl.BlockSpec((4 * RB,), lambda a, b: (a * NBLK + b,)),
        out_shape=jax.ShapeDtypeStruct((TOT,), jnp.int32),
    )(table_t)


@functools.partial(
    pl.kernel,
    mesh=_mesh,
    out_type=jax.ShapeDtypeStruct((B,), jnp.float32),
    compiler_params=pltpu.CompilerParams(needs_layout_passes=False),
    scratch_types=[
        pltpu.VMEM((C,), jnp.int32),
        pltpu.VMEM((C,), jnp.int32),
        pltpu.VMEM((Q, C), jnp.int32),
        pltpu.VMEM((Q, C), jnp.int32),
        pltpu.VMEM((Q * C,), jnp.int32),
        pltpu.VMEM((Q * C,), jnp.int32),
        pltpu.VMEM((B_PER_W,), jnp.float32),
        pltpu.SemaphoreType.DMA,
        pltpu.SemaphoreType.DMA,
    ],
)
def _dot_kernel(users_hbm, items_hbm, ue1d, ie1d, out_hbm,
                vidx_u, vidx_i, idxu2, idxi2, ut, it, outv, sem_u, sem_i):
    wid = lax.axis_index("s") * NC + lax.axis_index("c")
    base = wid * B_PER_W

    def chunk_body(k, carry):
        pltpu.sync_copy(users_hbm.at[pl.ds(base + k * C, C)], vidx_u)
        pltpu.sync_copy(items_hbm.at[pl.ds(base + k * C, C)], vidx_i)

        for h in range(C // L):
            uvec = vidx_u[pl.ds(h * L, L)]
            ivec = vidx_i[pl.ds(h * L, L)]
            ua = ((uvec >> 16) << 18) + (uvec & 0xFFFF)
            ia = ((ivec >> 16) << 18) + (ivec & 0xFFFF)
            for q in range(Q):
                aq = (q >> 2) * 4194304 + (q & 3) * 65536
                idxu2[q, pl.ds(h * L, L)] = ua + aq
                idxi2[q, pl.ds(h * L, L)] = ia + aq

        copies = []
        for q in range(Q):
            copies.append(pltpu.async_copy(
                ue1d.at[idxu2.at[q]], ut.at[pl.ds(q * C, C)], sem_u))
            copies.append(pltpu.async_copy(
                ie1d.at[idxi2.at[q]], it.at[pl.ds(q * C, C)], sem_i))
        for cp in copies:
            cp.wait()

        for g in range(C // L):
            acc = jnp.zeros((L,), jnp.float32)
            for q in range(Q):
                uw = plsc.bitcast(ut[pl.ds(q * C + g * L, L)], jnp.bfloat16)
                iw = plsc.bitcast(it[pl.ds(q * C + g * L, L)], jnp.bfloat16)
                ulo, uhi = plsc.unpack(uw, format=plsc.PackFormat.INTERLEAVED)
                ilo, ihi = plsc.unpack(iw, format=plsc.PackFormat.INTERLEAVED)
                acc = acc + ulo * ilo + uhi * ihi
            outv[pl.ds(k * C + g * L, L)] = acc
        return carry

    lax.fori_loop(0, B_PER_W // C, chunk_body, 0)

    pltpu.sync_copy(outv, out_hbm.at[pl.ds(base, B_PER_W)])


def kernel(users, items, user_embeddings, item_embeddings):
    ue1d = _linearize(jnp.swapaxes(user_embeddings, 0, 1))
    ie1d = _linearize(jnp.swapaxes(item_embeddings, 0, 1))
    out = _dot_kernel(users.astype(jnp.int32), items.astype(jnp.int32),
                      ue1d, ie1d)
    return out.reshape(B, 1)


# speedup vs baseline: 2.5485x; 2.5485x over previous
import functools

import jax
import jax.numpy as jnp
from jax import lax
from jax.experimental import pallas as pl
from jax.experimental.pallas import tpu as pltpu
from jax.experimental.pallas import tpu_sc as plsc

B = 16384
F = 32
L = 16
NC = 2
NS = 16
NW = NC * NS
B_PER_W = B // NW
C = 128
NROWS = 1000000
RB = 131072
NBLK = 8
TOT = 4 * NBLK * 8 * RB

_mesh = plsc.VectorSubcoreMesh(core_axis_name="c", subcore_axis_name="s")


def _linearize(table_t):
    def body(x_ref, o_ref):
        for s in range(8):
            o_ref[pl.ds(s * RB, RB)] = x_ref[s, :]

    return pl.pallas_call(
        body,
        grid=(4, NBLK),
        in_specs=[pl.BlockSpec((8, RB), lambda a, b: (a, b))],
        out_specs=pl.BlockSpec((8 * RB,), lambda a, b: (a * NBLK + b,)),
        out_shape=jax.ShapeDtypeStruct((TOT,), jnp.float32),
    )(table_t)


@functools.partial(
    pl.kernel,
    mesh=_mesh,
    out_type=jax.ShapeDtypeStruct((B,), jnp.float32),
    compiler_params=pltpu.CompilerParams(needs_layout_passes=False),
    scratch_types=[
        pltpu.VMEM((C,), jnp.int32),
        pltpu.VMEM((C,), jnp.int32),
        pltpu.VMEM((F, C), jnp.int32),
        pltpu.VMEM((F, C), jnp.int32),
        pltpu.VMEM((F, C), jnp.float32),
        pltpu.VMEM((F, C), jnp.float32),
        pltpu.VMEM((B_PER_W,), jnp.float32),
        pltpu.SemaphoreType.DMA,
        pltpu.SemaphoreType.DMA,
    ],
)
def _dot_kernel(users_hbm, items_hbm, ue1d, ie1d, out_hbm,
                vidx_u, vidx_i, idxu2, idxi2, ut, it, outv, sem_u, sem_i):
    wid = lax.axis_index("s") * NC + lax.axis_index("c")
    base = wid * B_PER_W

    def chunk_body(k, carry):
        pltpu.sync_copy(users_hbm.at[pl.ds(base + k * C, C)], vidx_u)
        pltpu.sync_copy(items_hbm.at[pl.ds(base + k * C, C)], vidx_i)

        for h in range(C // L):
            uvec = vidx_u[pl.ds(h * L, L)]
            ivec = vidx_i[pl.ds(h * L, L)]
            ua = ((uvec >> 17) << 20) + (uvec & 0x1FFFF)
            ia = ((ivec >> 17) << 20) + (ivec & 0x1FFFF)
            for f in range(F):
                af = (f >> 3) * 8388608 + (f & 7) * 131072
                idxu2[f, pl.ds(h * L, L)] = ua + af
                idxi2[f, pl.ds(h * L, L)] = ia + af

        copies = []
        for f in range(F):
            copies.append(pltpu.async_copy(
                ue1d.at[idxu2.at[f]], ut.at[f], sem_u))
            copies.append(pltpu.async_copy(
                ie1d.at[idxi2.at[f]], it.at[f], sem_i))
        for cp in copies:
            cp.wait()

        for g in range(C // L):
            acc = jnp.zeros((L,), jnp.float32)
            for f in range(F):
                acc = acc + ut[f, pl.ds(g * L, L)] * it[f, pl.ds(g * L, L)]
            outv[pl.ds(k * C + g * L, L)] = acc
        return carry

    lax.fori_loop(0, B_PER_W // C, chunk_body, 0)

    pltpu.sync_copy(outv, out_hbm.at[pl.ds(base, B_PER_W)])


def kernel(users, items, user_embeddings, item_embeddings):
    ue1d = _linearize(jnp.swapaxes(user_embeddings, 0, 1))
    ie1d = _linearize(jnp.swapaxes(item_embeddings, 0, 1))
    out = _dot_kernel(users.astype(jnp.int32), items.astype(jnp.int32),
                      ue1d, ie1d)
    return out.reshape(B, 1)


# speedup vs baseline: 2.5989x; 1.0198x over previous
import functools

import jax
import jax.numpy as jnp
from jax import lax
from jax.experimental import pallas as pl
from jax.experimental.pallas import tpu as pltpu
from jax.experimental.pallas import tpu_sc as plsc

B = 16384
F = 32
L = 16
NC = 2
NS = 16
NW = NC * NS
B_PER_W = B // NW
C = 128
NROWS = 1000000
RB = 262144
NBLK = 4
TOT = 4 * NBLK * 8 * RB

_mesh = plsc.VectorSubcoreMesh(core_axis_name="c", subcore_axis_name="s")


def _linearize(table_t):
    def body(x_ref, o_ref):
        for s in range(8):
            o_ref[pl.ds(s * RB, RB)] = x_ref[s, :]

    return pl.pallas_call(
        body,
        grid=(4, NBLK),
        in_specs=[pl.BlockSpec((8, RB), lambda a, b: (a, b))],
        out_specs=pl.BlockSpec((8 * RB,), lambda a, b: (a * NBLK + b,)),
        out_shape=jax.ShapeDtypeStruct((TOT,), jnp.float32),
    )(table_t)


@functools.partial(
    pl.kernel,
    mesh=_mesh,
    out_type=jax.ShapeDtypeStruct((B,), jnp.float32),
    compiler_params=pltpu.CompilerParams(needs_layout_passes=False),
    scratch_types=[
        pltpu.VMEM((C,), jnp.int32),
        pltpu.VMEM((C,), jnp.int32),
        pltpu.VMEM((F, C), jnp.int32),
        pltpu.VMEM((F, C), jnp.int32),
        pltpu.VMEM((F, C), jnp.float32),
        pltpu.VMEM((F, C), jnp.float32),
        pltpu.VMEM((B_PER_W,), jnp.float32),
        pltpu.SemaphoreType.DMA,
        pltpu.SemaphoreType.DMA,
    ],
)
def _dot_kernel(users_hbm, items_hbm, ue1d, ie1d, out_hbm,
                vidx_u, vidx_i, idxu2, idxi2, ut, it, outv, sem_u, sem_i):
    wid = lax.axis_index("s") * NC + lax.axis_index("c")
    base = wid * B_PER_W

    def chunk_body(k, carry):
        pltpu.sync_copy(users_hbm.at[pl.ds(base + k * C, C)], vidx_u)
        pltpu.sync_copy(items_hbm.at[pl.ds(base + k * C, C)], vidx_i)

        for h in range(C // L):
            uvec = vidx_u[pl.ds(h * L, L)]
            ivec = vidx_i[pl.ds(h * L, L)]
            ua = ((uvec >> 18) << 21) + (uvec & 0x3FFFF)
            ia = ((ivec >> 18) << 21) + (ivec & 0x3FFFF)
            for f in range(F):
                af = (f >> 3) * 8388608 + (f & 7) * 262144
                idxu2[f, pl.ds(h * L, L)] = ua + af
                idxi2[f, pl.ds(h * L, L)] = ia + af

        copies = []
        for f in range(F):
            copies.append(pltpu.async_copy(
                ue1d.at[idxu2.at[f]], ut.at[f], sem_u))
            copies.append(pltpu.async_copy(
                ie1d.at[idxi2.at[f]], it.at[f], sem_i))
        for cp in copies:
            cp.wait()

        for g in range(C // L):
            acc = jnp.zeros((L,), jnp.float32)
            for f in range(F):
                acc = acc + ut[f, pl.ds(g * L, L)] * it[f, pl.ds(g * L, L)]
            outv[pl.ds(k * C + g * L, L)] = acc
        return carry

    lax.fori_loop(0, B_PER_W // C, chunk_body, 0)

    pltpu.sync_copy(outv, out_hbm.at[pl.ds(base, B_PER_W)])


def kernel(users, items, user_embeddings, item_embeddings):
    ue1d = _linearize(jnp.swapaxes(user_embeddings, 0, 1))
    ie1d = _linearize(jnp.swapaxes(item_embeddings, 0, 1))
    out = _dot_kernel(users.astype(jnp.int32), items.astype(jnp.int32),
                      ue1d, ie1d)
    return out.reshape(B, 1)


# speedup vs baseline: 2.6061x; 1.0028x over previous
import functools

import jax
import jax.numpy as jnp
from jax import lax
from jax.experimental import pallas as pl
from jax.experimental.pallas import tpu as pltpu
from jax.experimental.pallas import tpu_sc as plsc

B = 16384
F = 32
L = 16
NC = 2
NS = 16
NW = NC * NS
B_PER_W = B // NW
C = 128
NROWS = 1000000
RB = 262144
NBLK = 4
TOT = 4 * NBLK * 8 * RB

_mesh = plsc.VectorSubcoreMesh(core_axis_name="c", subcore_axis_name="s")


def _linearize(table_t):
    def body(x_ref, o_ref):
        for s in range(8):
            o_ref[pl.ds(s * RB, RB)] = x_ref[s, :]

    return pl.pallas_call(
        body,
        grid=(4, NBLK),
        in_specs=[pl.BlockSpec((8, RB), lambda a, b: (a, b))],
        out_specs=pl.BlockSpec((8 * RB,), lambda a, b: (a * NBLK + b,)),
        out_shape=jax.ShapeDtypeStruct((TOT,), jnp.float32),
    )(table_t)


@functools.partial(
    pl.kernel,
    mesh=_mesh,
    out_type=jax.ShapeDtypeStruct((B,), jnp.float32),
    compiler_params=pltpu.CompilerParams(needs_layout_passes=False),
    scratch_types=[
        pltpu.VMEM((2, C), jnp.int32),
        pltpu.VMEM((2, C), jnp.int32),
        pltpu.VMEM((2, F, C), jnp.int32),
        pltpu.VMEM((2, F, C), jnp.int32),
        pltpu.VMEM((2, F, C), jnp.float32),
        pltpu.VMEM((2, F, C), jnp.float32),
        pltpu.VMEM((B_PER_W,), jnp.float32),
        pltpu.SemaphoreType.DMA,
        pltpu.SemaphoreType.DMA,
    ],
)
def _dot_kernel(users_hbm, items_hbm, ue1d, ie1d, out_hbm,
                vidx_u, vidx_i, idxu2, idxi2, ut, it, outv, sem_u, sem_i):
    wid = lax.axis_index("s") * NC + lax.axis_index("c")
    base = wid * B_PER_W
    NCHUNK = B_PER_W // C

    def fire(k, par):
        pltpu.sync_copy(users_hbm.at[pl.ds(base + k * C, C)], vidx_u.at[par])
        pltpu.sync_copy(items_hbm.at[pl.ds(base + k * C, C)], vidx_i.at[par])
        for h in range(C // L):
            uvec = vidx_u[par, pl.ds(h * L, L)]
            ivec = vidx_i[par, pl.ds(h * L, L)]
            ua = ((uvec >> 18) << 21) + (uvec & 0x3FFFF)
            ia = ((ivec >> 18) << 21) + (ivec & 0x3FFFF)
            for f in range(F):
                af = (f >> 3) * 8388608 + (f & 7) * 262144
                idxu2[par, f, pl.ds(h * L, L)] = ua + af
                idxi2[par, f, pl.ds(h * L, L)] = ia + af
        for f in range(F):
            pltpu.async_copy(ue1d.at[idxu2.at[par, f]], ut.at[par, f], sem_u)
            pltpu.async_copy(ie1d.at[idxi2.at[par, f]], it.at[par, f], sem_i)

    def drain_compute(k, par):
        for f in range(F):
            pltpu.make_async_copy(
                ue1d.at[idxu2.at[par, f]], ut.at[par, f], sem_u).wait()
            pltpu.make_async_copy(
                ie1d.at[idxi2.at[par, f]], it.at[par, f], sem_i).wait()
        for g in range(C // L):
            acc = jnp.zeros((L,), jnp.float32)
            for f in range(F):
                acc = acc + (ut[par, f, pl.ds(g * L, L)]
                             * it[par, f, pl.ds(g * L, L)])
            outv[pl.ds(k * C + g * L, L)] = acc

    fire(0, 0)

    def chunk_body(k, carry):
        fire(k, k % 2)
        drain_compute(k - 1, (k - 1) % 2)
        return carry

    lax.fori_loop(1, NCHUNK, chunk_body, 0)
    drain_compute(NCHUNK - 1, (NCHUNK - 1) % 2)

    pltpu.sync_copy(outv, out_hbm.at[pl.ds(base, B_PER_W)])


def kernel(users, items, user_embeddings, item_embeddings):
    ue1d = _linearize(jnp.swapaxes(user_embeddings, 0, 1))
    ie1d = _linearize(jnp.swapaxes(item_embeddings, 0, 1))
    out = _dot_kernel(users.astype(jnp.int32), items.astype(jnp.int32),
                      ue1d, ie1d)
    return out.reshape(B, 1)


# speedup vs baseline: 2.6182x; 1.0046x over previous
import functools

import jax
import jax.numpy as jnp
from jax import lax
from jax.experimental import pallas as pl
from jax.experimental.pallas import tpu as pltpu
from jax.experimental.pallas import tpu_sc as plsc

B = 16384
F = 32
L = 16
NC = 2
NS = 16
NW = NC * NS
B_PER_W = B // NW
C = 128
NROWS = 1000000
RB = 131072
NBLK = 8
TOT = 4 * NBLK * 8 * RB

_mesh = plsc.VectorSubcoreMesh(core_axis_name="c", subcore_axis_name="s")


def _linearize2(ut_t, it_t):
    def body(x_ref, y_ref, o_ref, p_ref):
        for s in range(8):
            o_ref[pl.ds(s * RB, RB)] = x_ref[s, :]
            p_ref[pl.ds(s * RB, RB)] = y_ref[s, :]

    spec_in = pl.BlockSpec((8, RB), lambda a, b: (a, b))
    spec_out = pl.BlockSpec((8 * RB,), lambda a, b: (a * NBLK + b,))
    return pl.pallas_call(
        body,
        grid=(4, NBLK),
        in_specs=[spec_in, spec_in],
        out_specs=[spec_out, spec_out],
        out_shape=[jax.ShapeDtypeStruct((TOT,), jnp.float32),
                   jax.ShapeDtypeStruct((TOT,), jnp.float32)],
    )(ut_t, it_t)


@functools.partial(
    pl.kernel,
    mesh=_mesh,
    out_type=jax.ShapeDtypeStruct((B,), jnp.float32),
    compiler_params=pltpu.CompilerParams(needs_layout_passes=False),
    scratch_types=[
        pltpu.VMEM((2, C), jnp.int32),
        pltpu.VMEM((2, C), jnp.int32),
        pltpu.VMEM((2, F, C), jnp.int32),
        pltpu.VMEM((2, F, C), jnp.int32),
        pltpu.VMEM((2, F, C), jnp.float32),
        pltpu.VMEM((2, F, C), jnp.float32),
        pltpu.VMEM((B_PER_W,), jnp.float32),
        pltpu.SemaphoreType.DMA,
        pltpu.SemaphoreType.DMA,
    ],
)
def _dot_kernel(users_hbm, items_hbm, ue1d, ie1d, out_hbm,
                vidx_u, vidx_i, idxu2, idxi2, ut, it, outv, sem_u, sem_i):
    wid = lax.axis_index("s") * NC + lax.axis_index("c")
    base = wid * B_PER_W
    NCHUNK = B_PER_W // C

    def fire(k, par):
        pltpu.sync_copy(users_hbm.at[pl.ds(base + k * C, C)], vidx_u.at[par])
        pltpu.sync_copy(items_hbm.at[pl.ds(base + k * C, C)], vidx_i.at[par])
        for h in range(C // L):
            uvec = vidx_u[par, pl.ds(h * L, L)]
            ivec = vidx_i[par, pl.ds(h * L, L)]
            ua = ((uvec >> 17) << 20) + (uvec & 0x1FFFF)
            ia = ((ivec >> 17) << 20) + (ivec & 0x1FFFF)
            for f in range(F):
                af = (f >> 3) * 8388608 + (f & 7) * 131072
                idxu2[par, f, pl.ds(h * L, L)] = ua + af
                idxi2[par, f, pl.ds(h * L, L)] = ia + af
        for f in range(F):
            pltpu.async_copy(ue1d.at[idxu2.at[par, f]], ut.at[par, f], sem_u)
            pltpu.async_copy(ie1d.at[idxi2.at[par, f]], it.at[par, f], sem_i)

    def drain_compute(k, par):
        for f in range(F):
            pltpu.make_async_copy(
                ue1d.at[idxu2.at[par, f]], ut.at[par, f], sem_u).wait()
            pltpu.make_async_copy(
                ie1d.at[idxi2.at[par, f]], it.at[par, f], sem_i).wait()
        for g in range(C // L):
            acc = jnp.zeros((L,), jnp.float32)
            for f in range(F):
                acc = acc + (ut[par, f, pl.ds(g * L, L)]
                             * it[par, f, pl.ds(g * L, L)])
            outv[pl.ds(k * C + g * L, L)] = acc

    fire(0, 0)

    def chunk_body(k, carry):
        fire(k, k % 2)
        drain_compute(k - 1, (k - 1) % 2)
        return carry

    lax.fori_loop(1, NCHUNK, chunk_body, 0)
    drain_compute(NCHUNK - 1, (NCHUNK - 1) % 2)

    pltpu.sync_copy(outv, out_hbm.at[pl.ds(base, B_PER_W)])


def kernel(users, items, user_embeddings, item_embeddings):
    ue1d, ie1d = _linearize2(jnp.swapaxes(user_embeddings, 0, 1),
                             jnp.swapaxes(item_embeddings, 0, 1))
    out = _dot_kernel(users.astype(jnp.int32), items.astype(jnp.int32),
                      ue1d, ie1d)
    return out.reshape(B, 1)
